# baseline (speedup 1.0000x reference)
import functools

import jax
import jax.numpy as jnp
from jax import lax
from jax.experimental import pallas as pl
from jax.experimental.pallas import tpu as pltpu
from jax.experimental.pallas import tpu_sc as plsc

_N = 10000
_E = 320000
_NPAD = 10240
_NATOM = 11
_ND = 10001

_NC, _NS, _LANES = 2, 16, 16
_NW = _NC * _NS
_CH = 100
_NCH = _E // (_NW * _CH)
_NCHH = _NCH // 2
_RPT = _NPAD // _NS

_BM = 512
_GRID = _NPAD // _BM

_f32 = jnp.float32

_sc_mesh = plsc.VectorSubcoreMesh(
    core_axis_name="c", subcore_axis_name="s",
    num_cores=_NC, num_subcores=_NS)
_lin_params = pltpu.CompilerParams(use_tc_tiling_on_sc=False)


def _fill_1d(buf, n, value):
    v = jnp.full((_LANES,), value, _f32)

    def body(i, _):
        buf[pl.ds(i * _LANES, _LANES)] = v
        return 0

    lax.fori_loop(0, n // _LANES, body, 0)


def _fill_2d(buf, rows, cols, value):
    v = jnp.full((_LANES,), value, _f32)
    nseg = cols // _LANES

    def body(i, _):
        for k in range(nseg):
            buf[i, pl.ds(k * _LANES, _LANES)] = v
        return 0

    lax.fori_loop(0, rows, body, 0)


def _zero_acc2d(gbuf, acc_s, s, cols):
    _fill_2d(gbuf, _CH, cols, 0.0)
    zsl = gbuf.at[pl.ds(0, 80)]
    for k in range(_RPT // 80):
        pltpu.sync_copy(zsl, acc_s.at[pl.ds(s * _RPT + k * 80, 80)])


class _Stream:
    def __init__(self, y_hbm, gidx, sidx2, acc_s, bufs, sems,
                 nch=_NCH):
        self.y_hbm = y_hbm
        self.gidx = gidx
        self.sidx = sidx2
        self.acc = acc_s
        self.b = bufs
        self.gs = sems
        self.depth = len(bufs)
        self.nch = nch

    def start(self, k, j):
        pltpu.async_copy(self.y_hbm.at[self.gidx.at[j]], self.b[k],
                         self.gs[k])

    def gwait(self, k):
        pltpu.make_async_copy(self.y_hbm.at[self.gidx.at[0]], self.b[k],
                              self.gs[k]).wait()

    def scat(self, k, j):
        pltpu.sync_copy(self.b[k], self.acc.at[self.sidx.at[j]],
                        add=True)

    def prime(self):
        for k in range(self.depth):
            self.start(k, k)

    def step(self, k, j):
        self.gwait(k)
        self.scat(k, j)

        @pl.when(j + self.depth < self.nch)
        def _():
            self.start(k, j + self.depth)


def _run_stream(st, nch):
    st.prime()
    d = st.depth

    def body(t, _):
        j = d * t
        for k in range(d):
            st.step(k, j + k)
        return 0

    lax.fori_loop(0, nch // d, body, 0)


def _deg_body(ep_hbm, el_hbm, outp_hbm, outl_hbm,
              idx_v, ones_v, zed_v, histp_s, histl_s, sem):
    c = lax.axis_index("c")
    s = lax.axis_index("s")
    w = c * _NS + s
    _fill_1d(zed_v, _RPT, 0.0)
    _fill_1d(ones_v, 112, 1.0)
    pltpu.sync_copy(zed_v, histp_s.at[pl.ds(s * _RPT, _RPT)])
    pltpu.sync_copy(zed_v, histl_s.at[pl.ds(s * _RPT, _RPT)])
    plsc.subcore_barrier()

    ones_sl = ones_v.at[pl.ds(0, _CH)]

    def scatter_ones(hist_s):
        def wait_one():
            pltpu.make_async_copy(ones_sl, hist_s.at[idx_v.at[0]],
                                  sem).wait()

        def body(j, _):
            @pl.when(j >= 5)
            def _():
                wait_one()

            pltpu.async_copy(ones_sl, hist_s.at[idx_v.at[j]], sem,
                             add=True)
            return 0

        lax.fori_loop(0, _NCH, body, 0)
        for _k in range(5):
            wait_one()

    pltpu.sync_copy(ep_hbm.at[1, w, 0], idx_v.at[pl.ds(0, _NCHH)])
    pltpu.sync_copy(ep_hbm.at[1, w, 1], idx_v.at[pl.ds(_NCHH, _NCHH)])
    scatter_ones(histp_s)
    pltpu.sync_copy(el_hbm.at[1, w, 0], idx_v.at[pl.ds(0, _NCHH)])
    pltpu.sync_copy(el_hbm.at[1, w, 1], idx_v.at[pl.ds(_NCHH, _NCHH)])
    scatter_ones(histl_s)
    plsc.subcore_barrier()
    sl = pl.ds(s * _RPT, _RPT)
    pltpu.sync_copy(histp_s.at[sl], outp_hbm.at[c, sl])
    pltpu.sync_copy(histl_s.at[sl], outl_hbm.at[c, sl])


_deg_call = pl.kernel(
    _deg_body,
    out_type=[jax.ShapeDtypeStruct((_NC, _NPAD), _f32),
              jax.ShapeDtypeStruct((_NC, _NPAD), _f32)],
    mesh=_sc_mesh,
    compiler_params=_lin_params,
    scratch_types=[
        pltpu.VMEM((_NCH, _CH), jnp.int32),
        pltpu.VMEM((112,), _f32),
        pltpu.VMEM((_RPT,), _f32),
        pltpu.VMEM_SHARED((_NPAD,), _f32),
        pltpu.VMEM_SHARED((_NPAD,), _f32),
        pltpu.SemaphoreType.DMA,
    ])


def _spmv_body(y_hbm, e_hbm, out_hbm,
               sidx, didx, b0, b1, acc_s, s0, s1):
    c = lax.axis_index("c")
    s = lax.axis_index("s")
    w = c * _NS + s
    _zero_acc2d(b0, acc_s, s, 128)
    plsc.subcore_barrier()
    st = _Stream(y_hbm, sidx, didx, acc_s, (b0, b1), (s0, s1),
                 nch=_NCHH)
    for p in range(2):
        pltpu.sync_copy(e_hbm.at[0, w, p], sidx)
        pltpu.sync_copy(e_hbm.at[1, w, p], didx)
        _run_stream(st, _NCHH)
    plsc.subcore_barrier()
    sl = pl.ds(s * _RPT, _RPT)
    pltpu.sync_copy(acc_s.at[sl], out_hbm.at[c, sl])


_spmv_call = pl.kernel(
    _spmv_body,
    out_type=[jax.ShapeDtypeStruct((_NC, _NPAD, 128), _f32)],
    mesh=_sc_mesh,
    scratch_types=[
        pltpu.VMEM((_NCHH, _CH), jnp.int32),
        pltpu.VMEM((_NCHH, _CH), jnp.int32),
        pltpu.VMEM((_CH, 128), _f32),
        pltpu.VMEM((_CH, 128), _f32),
        pltpu.VMEM_SHARED((_NPAD, 128), _f32),
        pltpu.SemaphoreType.DMA,
        pltpu.SemaphoreType.DMA,
    ])


def _midl_body(inv_hbm, yl_hbm, ep_hbm, el_hbm, g_hbm, aggl_hbm,
               gsidx, gdidx, lsidx, ldidx,
               vb0, vb1, vb2, vb3, lb0, lb1, lb2, lb3,
               gacc_s, accl_s,
               gs0, gs1, gs2, gs3, ls0, ls1, ls2, ls3):
    c = lax.axis_index("c")
    s = lax.axis_index("s")
    w = c * _NS + s
    _zero_acc2d(lb0, accl_s, s, 16)
    _fill_1d(vb0, 96, 0.0)
    zvs = vb0.at[pl.ds(0, 80)]
    for k in range(_RPT // 80):
        pltpu.sync_copy(zvs, gacc_s.at[pl.ds(s * _RPT + k * 80, 80)])
    plsc.subcore_barrier()

    for q, buf in ((0, gsidx), (1, gdidx)):
        pltpu.sync_copy(ep_hbm.at[q, w, 0], buf.at[pl.ds(0, _NCHH)])
        pltpu.sync_copy(ep_hbm.at[q, w, 1], buf.at[pl.ds(_NCHH, _NCHH)])
    for q, buf in ((0, lsidx), (1, ldidx)):
        pltpu.sync_copy(el_hbm.at[q, w, 0], buf.at[pl.ds(0, _NCHH)])
        pltpu.sync_copy(el_hbm.at[q, w, 1], buf.at[pl.ds(_NCHH, _NCHH)])

    G = _Stream(inv_hbm, gdidx, gsidx, gacc_s,
                (vb0, vb1, vb2, vb3), (gs0, gs1, gs2, gs3))
    L = _Stream(yl_hbm, lsidx, ldidx, accl_s,
                (lb0, lb1, lb2, lb3), (ls0, ls1, ls2, ls3))
    G.prime()
    L.prime()

    def body(t, _):
        j = 4 * t
        for k in range(4):
            G.step(k, j + k)
            L.step(k, j + k)
        return 0

    lax.fori_loop(0, _NCH // 4, body, 0)
    plsc.subcore_barrier()
    sl = pl.ds(s * _RPT, _RPT)
    pltpu.sync_copy(gacc_s.at[sl], g_hbm.at[c, sl])
    pltpu.sync_copy(accl_s.at[sl], aggl_hbm.at[c, sl])


_midl_call = pl.kernel(
    _midl_body,
    out_type=[jax.ShapeDtypeStruct((_NC, _NPAD), _f32),
              jax.ShapeDtypeStruct((_NC, _NPAD, 16), _f32)],
    mesh=_sc_mesh,
    compiler_params=_lin_params,
    scratch_types=[
        pltpu.VMEM((_NCH, _CH), jnp.int32),
        pltpu.VMEM((_NCH, _CH), jnp.int32),
        pltpu.VMEM((_NCH, _CH), jnp.int32),
        pltpu.VMEM((_NCH, _CH), jnp.int32),
        pltpu.VMEM((_CH,), _f32),
        pltpu.VMEM((_CH,), _f32),
        pltpu.VMEM((_CH,), _f32),
        pltpu.VMEM((_CH,), _f32),
        pltpu.VMEM((_CH, 16), _f32),
        pltpu.VMEM((_CH, 16), _f32),
        pltpu.VMEM((_CH, 16), _f32),
        pltpu.VMEM((_CH, 16), _f32),
        pltpu.VMEM_SHARED((_NPAD,), _f32),
        pltpu.VMEM_SHARED((_NPAD, 16), _f32),
    ] + [pltpu.SemaphoreType.DMA] * 8)


def _prep_body(degp_ref, degl_ref, xp_ref, xl_ref,
               yp_ref, yl_ref, invp_ref, invl_ref):
    invp = lax.rsqrt(degp_ref[...])
    invl = lax.rsqrt(degl_ref[...])
    invp_ref[...] = invp
    invl_ref[...] = invl
    yp_ref[...] = xp_ref[...] * invp
    xl16 = jnp.concatenate(
        [xl_ref[...], jnp.zeros((_BM, 1), _f32)], axis=1)
    yl_ref[...] = xl16 * invl


def _prep_call(degp, degl, x_p, x_l):
    blk = lambda r, c: pl.BlockSpec((r, c), lambda i: (i, 0))
    return pl.pallas_call(
        _prep_body,
        grid=(_GRID,),
        in_specs=[blk(_BM, 1), blk(_BM, 1), blk(_BM, 128), blk(_BM, 15)],
        out_specs=[blk(_BM, 128), blk(_BM, 16), blk(_BM, 1),
                   blk(_BM, 1)],
        out_shape=[jax.ShapeDtypeStruct((_NPAD, 128), _f32),
                   jax.ShapeDtypeStruct((_NPAD, 16), _f32),
                   jax.ShapeDtypeStruct((_NPAD, 1), _f32),
                   jax.ShapeDtypeStruct((_NPAD, 1), _f32)],
    )(degp, degl, x_p, x_l)


def _mid_tc_body(ap_ref, yp_ref, invp_ref, g_ref,
                 al_ref, yl_ref, invl_ref,
                 wp1_ref, bp1_ref, wp2_ref, bp2_ref, wl1_ref, bl1_ref,
                 y2_ref, zp_ref, sacc_ref):
    i = pl.program_id(0)
    invp = invp_ref[...]
    aggp = invp * (ap_ref[0] + ap_ref[1] + yp_ref[...])
    h = jnp.maximum(
        jnp.dot(aggp, wp1_ref[...], preferred_element_type=_f32)
        + bp1_ref[...], 0.0)
    rowid = lax.broadcasted_iota(jnp.int32, (_BM, 1), 0) + i * _BM
    cvec = jnp.where(rowid < _N, invp * (g_ref[...] + invp), 0.0)

    @pl.when(i == 0)
    def _():
        sacc_ref[...] = jnp.zeros_like(sacc_ref)

    sacc_ref[...] += jnp.sum(cvec * h, axis=0, keepdims=True)

    invl = invl_ref[...]
    aggl = invl * (al_ref[0] + al_ref[1] + yl_ref[...])
    wl1p = jnp.concatenate([wl1_ref[...], jnp.zeros((1, 128), _f32)],
                           axis=0)
    hl = jnp.maximum(
        jnp.dot(aggl, wl1p, preferred_element_type=_f32)
        + bl1_ref[...], 0.0)
    y2_ref[...] = invl * hl

    @pl.when(i == _GRID - 1)
    def _():
        zp = jnp.dot(sacc_ref[...] / _N, wp2_ref[...],
                     preferred_element_type=_f32) + bp2_ref[...]
        zp_ref[...] = zp[0]


def _mid_tc_call(aggp, yp, invp, gcol, aggl, yl, invl,
                 Wp1, bp1, Wp2, bp2, Wl1, bl1):
    blk = lambda r, c: pl.BlockSpec((r, c), lambda i: (i, 0))
    blk3 = lambda c: pl.BlockSpec((_NC, _BM, c), lambda i: (0, i, 0))
    cst = lambda r, c: pl.BlockSpec((r, c), lambda i: (0, 0))
    return pl.pallas_call(
        _mid_tc_body,
        grid=(_GRID,),
        in_specs=[blk3(128), blk(_BM, 128), blk(_BM, 1), blk(_BM, 1),
                  blk3(16), blk(_BM, 16), blk(_BM, 1),
                  cst(128, 128), cst(1, 128), cst(128, 128), cst(1, 128),
                  cst(15, 128), cst(1, 128)],
        out_specs=[blk(_BM, 128),
                   pl.BlockSpec((128,), lambda i: (0,))],
        out_shape=[jax.ShapeDtypeStruct((_NPAD, 128), _f32),
                   jax.ShapeDtypeStruct((128,), _f32)],
        scratch_shapes=[pltpu.VMEM((1, 128), _f32)],
    )(aggp, yp, invp, gcol, aggl, yl, invl,
      Wp1, bp1[None, :], Wp2, bp2[None, :], Wl1, bl1[None, :])


def _dec_body(bfs_ref, xl_ref, xlf_ref,
              wd1_ref, bd1_ref, wd2_ref, bd2_ref,
              zv_ref, ht_ref, zvsum_ref, labB_ref):
    i = pl.program_id(0)
    rowid = lax.broadcasted_iota(jnp.int32, (_BM, 1), 0) + i * _BM

    @pl.when(i == 0)
    def _():
        zvsum_ref[...] = jnp.zeros_like(zvsum_ref)
        labB_ref[...] = jnp.zeros_like(labB_ref)

    stop_row = jnp.where(
        lax.broadcasted_iota(jnp.int32, (1, _NATOM), 1) == _NATOM - 1,
        1.0, 0.0)
    lab = jnp.where(rowid < _N, xl_ref[...][:, 4:], 0.0)
    lab = jnp.where(rowid == _N, stop_row, lab)
    labB_ref[...] += jnp.sum(jnp.where(rowid < _ND, lab, 0.0), axis=0,
                             keepdims=True)

    src0 = bfs_ref[0, 0]
    dst0 = bfs_ref[1, 0]
    same = src0 == dst0
    isq = 0.70710678118654752
    x_s0 = xlf_ref[pl.ds(src0, 1), :][:, 4:]
    x_d0 = xlf_ref[pl.ds(dst0, 1), :][:, 4:]
    agg_d = jnp.where(same, x_d0, isq * x_s0 + 0.5 * x_d0)
    h_spec = jnp.maximum(
        jnp.dot(agg_d, wd1_ref[...], preferred_element_type=_f32)
        + bd1_ref[...], 0.0)
    h_src0 = jnp.maximum(
        jnp.dot(x_s0, wd1_ref[...], preferred_element_type=_f32)
        + bd1_ref[...], 0.0)
    agg2_d = jnp.where(same, h_spec, isq * h_src0 + 0.5 * h_spec)
    z_spec = jnp.dot(agg2_d, wd2_ref[...], preferred_element_type=_f32) \
        + bd2_ref[...]
    h = jnp.maximum(
        jnp.dot(lab, wd1_ref[...], preferred_element_type=_f32)
        + bd1_ref[...], 0.0)
    h = jnp.where(rowid == dst0, h_spec, h)
    zv = jnp.dot(h, wd2_ref[...], preferred_element_type=_f32) \
        + bd2_ref[...]
    zv = jnp.where(rowid == dst0, z_spec, zv)
    zv_ref[...] = zv
    zvsum_ref[...] += jnp.sum(jnp.where(rowid < _ND, zv, 0.0), axis=0,
                              keepdims=True)

    @pl.when(i == _GRID - 1)
    def _():
        ht = jnp.concatenate([zvsum_ref[...] / _ND, labB_ref[...] / _ND],
                             axis=1)
        ht_ref[...] = ht[0]


def _dec_call(bfs, x_l, Wd1, bd1, Wd2, bd2):
    blk = lambda r, c: pl.BlockSpec((r, c), lambda i: (i, 0))
    cst = lambda r, c: pl.BlockSpec((r, c), lambda i: (0, 0))
    return pl.pallas_call(
        _dec_body,
        grid=(_GRID,),
        in_specs=[pl.BlockSpec(memory_space=pltpu.SMEM),
                  blk(_BM, 15),
                  cst(_N, 15),
                  cst(_NATOM, 128), cst(1, 128), cst(128, 128),
                  cst(1, 128)],
        out_specs=[blk(_BM, 128),
                   pl.BlockSpec((139,), lambda i: (0,))],
        out_shape=[jax.ShapeDtypeStruct((_ND, 128), _f32),
                   jax.ShapeDtypeStruct((139,), _f32)],
        scratch_shapes=[pltpu.VMEM((1, 128), _f32),
                        pltpu.VMEM((1, _NATOM), _f32)],
    )(bfs, x_l, x_l, Wd1, bd1[None, :], Wd2, bd2[None, :])


def _lig_body(q_ref, y2_ref, invl_ref, xl_ref,
              wl2_ref, bl2_ref, wf_ref, bf_ref,
              lp_ref, hi_ref, zlsum_ref, labA_ref):
    i = pl.program_id(0)
    rowid = lax.broadcasted_iota(jnp.int32, (_BM, 1), 0) + i * _BM

    @pl.when(i == 0)
    def _():
        lp_ref[...] = jnp.zeros_like(lp_ref)
        zlsum_ref[...] = jnp.zeros_like(zlsum_ref)
        labA_ref[...] = jnp.zeros_like(labA_ref)

    lab = jnp.where(rowid < _N, xl_ref[...][:, 4:], 0.0)
    invl = invl_ref[...]
    aggl2 = invl * (q_ref[0] + q_ref[1] + y2_ref[...])
    zl = jnp.dot(aggl2, wl2_ref[...], preferred_element_type=_f32) \
        + bl2_ref[...]
    lmask = jnp.where(
        lax.broadcasted_iota(jnp.int32, (1, _NATOM), 1) == _NATOM - 1,
        -1e9, 0.0)
    logits = jnp.dot(zl, wf_ref[...], preferred_element_type=_f32) \
        + bf_ref[...] + lmask
    m = jnp.max(logits, axis=1, keepdims=True)
    e = jnp.exp(logits - m)
    num = jnp.sum(e * lab, axis=1, keepdims=True)
    den = jnp.sum(e, axis=1, keepdims=True)
    lig_mask = rowid < _N
    inner = jnp.where(lig_mask, num / den, 1.0)
    lp_ref[...] += jnp.sum(jnp.log(inner), axis=0, keepdims=True)
    zlsum_ref[...] += jnp.sum(jnp.where(lig_mask, zl, 0.0), axis=0,
                              keepdims=True)
    labA_ref[...] += jnp.sum(jnp.where(lig_mask, lab, 0.0), axis=0,
                             keepdims=True)

    @pl.when(i == _GRID - 1)
    def _():
        hi = jnp.concatenate([zlsum_ref[...] / _N, labA_ref[...] / _N],
                             axis=1)
        hi_ref[...] = hi[0]


def _lig_call(aggl2, y2, invl, x_l, Wl2, bl2, Wf, bf):
    blk = lambda r, c: pl.BlockSpec((r, c), lambda i: (i, 0))
    blk3 = lambda c: pl.BlockSpec((_NC, _BM, c), lambda i: (0, i, 0))
    cst = lambda r, c: pl.BlockSpec((r, c), lambda i: (0, 0))
    return pl.pallas_call(
        _lig_body,
        grid=(_GRID,),
        in_specs=[blk3(128), blk(_BM, 128), blk(_BM, 1), blk(_BM, 15),
                  cst(128, 128), cst(1, 128), cst(128, _NATOM),
                  cst(1, _NATOM)],
        out_specs=[cst(1, 1),
                   pl.BlockSpec((139,), lambda i: (0,))],
        out_shape=[jax.ShapeDtypeStruct((1, 1), _f32),
                   jax.ShapeDtypeStruct((139,), _f32)],
        scratch_shapes=[pltpu.VMEM((1, 128), _f32),
                        pltpu.VMEM((1, _NATOM), _f32)],
    )(aggl2, y2, invl, x_l, Wl2, bl2[None, :], Wf, bf[None, :])


def kernel(x_p, edge_index_p, x_l, edge_index_l, bfs_init, Wp1, bp1, Wp2,
           bp2, Wl1, bl1, Wl2, bl2, Wd1, bd1, Wd2, bd2, Wf, bf):
    ep = edge_index_p.reshape(2, _NW, 2, _NCHH, _CH)
    el = edge_index_l.reshape(2, _NW, 2, _NCHH, _CH)

    z_v, H_t = _dec_call(bfs_init, x_l, Wd1, bd1, Wd2, bd2)

    dp, dl = _deg_call(ep, el)
    degp = (dp[0] + dp[1] + 1.0)[:, None]
    degl = (dl[0] + dl[1] + 1.0)[:, None]

    yp, yl, invp, invl = _prep_call(degp, degl, x_p, x_l)

    gmat, aggl = _midl_call(invp.reshape(_NPAD), yl, ep, el)
    (aggp,) = _spmv_call(yp, ep)
    gcol = (gmat[0] + gmat[1])[:, None]

    y2, z_pocket = _mid_tc_call(aggp, yp, invp, gcol, aggl, yl, invl,
                                Wp1, bp1, Wp2, bp2, Wl1, bl1)

    (aggl2,) = _spmv_call(y2, el)

    lp, H_init = _lig_call(aggl2, y2, invl, x_l, Wl2, bl2, Wf, bf)

    return (lp[0, 0], z_pocket, z_v, H_init, H_t)


# speedup vs baseline: 1.0789x; 1.0789x over previous
import functools

import jax
import jax.numpy as jnp
from jax import lax
from jax.experimental import pallas as pl
from jax.experimental.pallas import tpu as pltpu
from jax.experimental.pallas import tpu_sc as plsc

_N = 10000
_E = 320000
_NPAD = 10240
_NATOM = 11
_ND = 10001

_NC, _NS, _LANES = 2, 16, 16
_CH = 100
_NCHL = _E // (_NC * _NS * _CH)
_NCHP = _E // (_NS * _CH)
_RPT = _NPAD // _NS
_HD = 64

_BM = 512
_GRID = _NPAD // _BM

_f32 = jnp.float32

_sc_mesh = plsc.VectorSubcoreMesh(
    core_axis_name="c", subcore_axis_name="s",
    num_cores=_NC, num_subcores=_NS)
_sc_params = pltpu.CompilerParams(use_tc_tiling_on_sc=False)


def _fill_1d(buf, n, value):
    v = jnp.full((_LANES,), value, _f32)

    def body(i, _):
        buf[pl.ds(i * _LANES, _LANES)] = v
        return 0

    lax.fori_loop(0, n // _LANES, body, 0)


def _fill_2d(buf, rows, cols, value):
    v = jnp.full((_LANES,), value, _f32)
    nseg = cols // _LANES

    def body(i, _):
        for k in range(nseg):
            buf[i, pl.ds(k * _LANES, _LANES)] = v
        return 0

    lax.fori_loop(0, rows, body, 0)


class _Stream:
    def __init__(self, y_hbm, gidx, sidx2, acc_s, b0, b1, gs0, gs1):
        self.y_hbm = y_hbm
        self.gidx = gidx
        self.sidx = sidx2
        self.acc = acc_s
        self.b = (b0, b1)
        self.gs = (gs0, gs1)

    def start(self, k, j):
        pltpu.async_copy(self.y_hbm.at[self.gidx.at[j]], self.b[k],
                         self.gs[k])

    def gwait(self, k):
        pltpu.make_async_copy(self.y_hbm.at[pl.ds(0, _CH)], self.b[k],
                              self.gs[k]).wait()

    def scat(self, k, j):
        pltpu.sync_copy(self.b[k], self.acc.at[self.sidx.at[j]],
                        add=True)


def _run_stream(st, nch, base):
    st.start(0, base)
    st.start(1, base + 1)

    def body(t, _):
        j = base + 2 * t
        st.gwait(0)
        st.scat(0, j)

        @pl.when(j + 2 < base + nch)
        def _():
            st.start(0, j + 2)

        st.gwait(1)
        st.scat(1, j + 1)

        @pl.when(j + 3 < base + nch)
        def _():
            st.start(1, j + 3)

        return 0

    lax.fori_loop(0, nch // 2, body, 0)


class _Stream4:
    def __init__(self, y_hbm, gidx, sidx2, acc_s, bufs, sems):
        self.y_hbm = y_hbm
        self.gidx = gidx
        self.sidx = sidx2
        self.acc = acc_s
        self.b = bufs
        self.gs = sems

    def start(self, k, j):
        pltpu.async_copy(self.y_hbm.at[self.gidx.at[j]], self.b[k],
                         self.gs[k])

    def gwait(self, k):
        pltpu.make_async_copy(self.y_hbm.at[pl.ds(0, _CH)], self.b[k],
                              self.gs[k]).wait()

    def scat(self, k, j):
        pltpu.sync_copy(self.b[k], self.acc.at[self.sidx.at[j]],
                        add=True)


def _run_stream4(st, nch):
    for k in range(4):
        st.start(k, k)

    def body(t, _):
        j = 4 * t
        for k in range(4):
            st.gwait(k)
            st.scat(k, j + k)

            @pl.when(j + k + 4 < nch)
            def _():
                st.start(k, j + k + 4)

        return 0

    lax.fori_loop(0, nch // 4, body, 0)


def _deg_body(ep_hbm, el_hbm, outp_hbm, outl_hbm,
              idx_v, ones_v, zed_v, histp_s, histl_s, sem):
    c = lax.axis_index("c")
    s = lax.axis_index("s")
    _fill_1d(zed_v, _RPT, 0.0)
    _fill_1d(ones_v, 112, 1.0)
    pltpu.sync_copy(zed_v, histp_s.at[pl.ds(s * _RPT, _RPT)])
    pltpu.sync_copy(zed_v, histl_s.at[pl.ds(s * _RPT, _RPT)])
    plsc.subcore_barrier()

    ones_sl = ones_v.at[pl.ds(0, _CH)]

    def scatter_ones(hist_s):
        def wait_one():
            pltpu.make_async_copy(ones_sl, hist_s.at[idx_v.at[0]],
                                  sem).wait()

        def body(j, _):
            @pl.when(j >= 5)
            def _():
                wait_one()

            pltpu.async_copy(ones_sl, hist_s.at[idx_v.at[j]], sem,
                             add=True)
            return 0

        lax.fori_loop(0, _NCHL, body, 0)
        for _k in range(5):
            wait_one()

    pltpu.sync_copy(ep_hbm.at[1, s, pl.ds(c * _NCHL, _NCHL)], idx_v)
    scatter_ones(histp_s)
    pltpu.sync_copy(el_hbm.at[1, s, pl.ds(c * _NCHL, _NCHL)], idx_v)
    scatter_ones(histl_s)
    plsc.subcore_barrier()
    sl = pl.ds(s * _RPT, _RPT)
    pltpu.sync_copy(histp_s.at[sl], outp_hbm.at[c, sl])
    pltpu.sync_copy(histl_s.at[sl], outl_hbm.at[c, sl])


_deg_call = pl.kernel(
    _deg_body,
    out_type=[jax.ShapeDtypeStruct((_NC, _NPAD), _f32),
              jax.ShapeDtypeStruct((_NC, _NPAD), _f32)],
    mesh=_sc_mesh,
    compiler_params=_sc_params,
    scratch_types=[
        pltpu.VMEM((_NCHL, _CH), jnp.int32),
        pltpu.VMEM((112,), _f32),
        pltpu.VMEM((_RPT,), _f32),
        pltpu.VMEM_SHARED((_NPAD,), _f32),
        pltpu.VMEM_SHARED((_NPAD,), _f32),
        pltpu.SemaphoreType.DMA,
    ])


def _zero_acc2d(gbuf, acc_s, s, cols):
    _fill_2d(gbuf, _CH, cols, 0.0)
    zsl = gbuf.at[pl.ds(0, 80)]
    for k in range(_RPT // 80):
        pltpu.sync_copy(zsl, acc_s.at[pl.ds(s * _RPT + k * 80, 80)])


def _mid_body(yp_hbm, ep_hbm, inv_hbm, yl_hbm, el_hbm,
              aggp_hbm, g_hbm, aggl_hbm,
              sidx, didx, lsidx, ldidx,
              rb0, rb1, rb2, rb3, vb0, vb1, vb2, vb3,
              lb0, lb1, lb2, lb3,
              accp_s, gacc_s, accl_s,
              rs0, rs1, rs2, rs3, gs0, gs1, gs2, gs3,
              ls0, ls1, ls2, ls3):
    c = lax.axis_index("c")
    s = lax.axis_index("s")
    _zero_acc2d(rb0, accp_s, s, _HD)
    _zero_acc2d(lb0, accl_s, s, 16)
    _fill_1d(vb0, 96, 0.0)
    zvs = vb0.at[pl.ds(0, 80)]
    for k in range(_RPT // 80):
        pltpu.sync_copy(zvs, gacc_s.at[pl.ds(s * _RPT + k * 80, 80)])
    plsc.subcore_barrier()

    pltpu.sync_copy(el_hbm.at[0, s, pl.ds(c * _NCHL, _NCHL)], lsidx)
    pltpu.sync_copy(el_hbm.at[1, s, pl.ds(c * _NCHL, _NCHL)], ldidx)

    R = _Stream4(yp_hbm.at[c], sidx, didx, accp_s,
                 (rb0, rb1, rb2, rb3), (rs0, rs1, rs2, rs3))
    G = _Stream4(inv_hbm, didx, sidx, gacc_s,
                 (vb0, vb1, vb2, vb3), (gs0, gs1, gs2, gs3))
    L = _Stream4(yl_hbm, lsidx, ldidx, accl_s,
                 (lb0, lb1, lb2, lb3), (ls0, ls1, ls2, ls3))

    for p in range(2):
        pltpu.sync_copy(ep_hbm.at[0, s, pl.ds(p * _NCHL, _NCHL)], sidx)
        pltpu.sync_copy(ep_hbm.at[1, s, pl.ds(p * _NCHL, _NCHL)], didx)
        for k in range(4):
            R.start(k, k)

        @pl.when(c == p)
        def _():
            for k in range(4):
                G.start(k, k)

        if p == 0:
            for k in range(4):
                L.start(k, k)

        def body(t, _):
            j = 4 * t
            for k in range(4):
                R.gwait(k)
                R.scat(k, j + k)

                @pl.when(j + k + 4 < _NCHL)
                def _():
                    R.start(k, j + k + 4)

                @pl.when(c == p)
                def _():
                    G.gwait(k)
                    G.scat(k, j + k)

                    @pl.when(j + k + 4 < _NCHL)
                    def _():
                        G.start(k, j + k + 4)

                if p == 0:
                    L.gwait(k)
                    L.scat(k, j + k)

                    @pl.when(j + k + 4 < _NCHL)
                    def _():
                        L.start(k, j + k + 4)

            return 0

        lax.fori_loop(0, _NCHL // 4, body, 0)

    plsc.subcore_barrier()
    sl = pl.ds(s * _RPT, _RPT)
    pltpu.sync_copy(accp_s.at[sl], aggp_hbm.at[c, sl])
    pltpu.sync_copy(gacc_s.at[sl], g_hbm.at[c, sl])
    pltpu.sync_copy(accl_s.at[sl], aggl_hbm.at[c, sl])


_mid_call = pl.kernel(
    _mid_body,
    out_type=[jax.ShapeDtypeStruct((_NC, _NPAD, _HD), _f32),
              jax.ShapeDtypeStruct((_NC, _NPAD), _f32),
              jax.ShapeDtypeStruct((_NC, _NPAD, 16), _f32)],
    mesh=_sc_mesh,
    compiler_params=_sc_params,
    scratch_types=[
        pltpu.VMEM((_NCHL, _CH), jnp.int32),
        pltpu.VMEM((_NCHL, _CH), jnp.int32),
        pltpu.VMEM((_NCHL, _CH), jnp.int32),
        pltpu.VMEM((_NCHL, _CH), jnp.int32),
        pltpu.VMEM((_CH, _HD), _f32),
        pltpu.VMEM((_CH, _HD), _f32),
        pltpu.VMEM((_CH, _HD), _f32),
        pltpu.VMEM((_CH, _HD), _f32),
        pltpu.VMEM((_CH,), _f32),
        pltpu.VMEM((_CH,), _f32),
        pltpu.VMEM((_CH,), _f32),
        pltpu.VMEM((_CH,), _f32),
        pltpu.VMEM((_CH, 16), _f32),
        pltpu.VMEM((_CH, 16), _f32),
        pltpu.VMEM((_CH, 16), _f32),
        pltpu.VMEM((_CH, 16), _f32),
        pltpu.VMEM_SHARED((_NPAD, _HD), _f32),
        pltpu.VMEM_SHARED((_NPAD,), _f32),
        pltpu.VMEM_SHARED((_NPAD, 16), _f32),
    ] + [pltpu.SemaphoreType.DMA] * 12)


def _l2_body(y2_hbm, el_hbm, agg_hbm,
             sidx, didx, gb0, gb1, gb2, gb3, acc_s, gs0, gs1, gs2, gs3):
    c = lax.axis_index("c")
    s = lax.axis_index("s")
    _zero_acc2d(gb0, acc_s, s, _HD)
    plsc.subcore_barrier()
    pltpu.sync_copy(el_hbm.at[0, s], sidx)
    pltpu.sync_copy(el_hbm.at[1, s], didx)
    st = _Stream4(y2_hbm.at[c], sidx, didx, acc_s,
                  (gb0, gb1, gb2, gb3), (gs0, gs1, gs2, gs3))
    _run_stream4(st, _NCHP)
    plsc.subcore_barrier()
    sl = pl.ds(s * _RPT, _RPT)
    pltpu.sync_copy(acc_s.at[sl], agg_hbm.at[c, sl])


_l2_call = pl.kernel(
    _l2_body,
    out_type=[jax.ShapeDtypeStruct((_NC, _NPAD, _HD), _f32)],
    mesh=_sc_mesh,
    compiler_params=_sc_params,
    scratch_types=[
        pltpu.VMEM((_NCHP, _CH), jnp.int32),
        pltpu.VMEM((_NCHP, _CH), jnp.int32),
        pltpu.VMEM((_CH, _HD), _f32),
        pltpu.VMEM((_CH, _HD), _f32),
        pltpu.VMEM((_CH, _HD), _f32),
        pltpu.VMEM((_CH, _HD), _f32),
        pltpu.VMEM_SHARED((_NPAD, _HD), _f32),
        pltpu.SemaphoreType.DMA,
        pltpu.SemaphoreType.DMA,
        pltpu.SemaphoreType.DMA,
        pltpu.SemaphoreType.DMA,
    ])


def _prep_body(degp_ref, degl_ref, xp_ref, xl_ref,
               yp_ref, yl_ref, invp_ref, invl_ref):
    invp = lax.rsqrt(degp_ref[...])
    invl = lax.rsqrt(degl_ref[...])
    invp_ref[...] = invp
    invl_ref[...] = invl
    yp = xp_ref[...] * invp
    yp_ref[0, :, :] = yp[:, :_HD]
    yp_ref[1, :, :] = yp[:, _HD:]
    xl16 = jnp.concatenate(
        [xl_ref[...], jnp.zeros((_BM, 1), _f32)], axis=1)
    yl_ref[...] = xl16 * invl


def _prep_call(degp, degl, x_p, x_l):
    blkc = lambda c: pl.BlockSpec((_NC, _BM, c), lambda i: (0, i, 0))
    return pl.pallas_call(
        _prep_body,
        grid=(_GRID,),
        in_specs=[pl.BlockSpec((_BM, 1), lambda i: (i, 0)),
                  pl.BlockSpec((_BM, 1), lambda i: (i, 0)),
                  pl.BlockSpec((_BM, 128), lambda i: (i, 0)),
                  pl.BlockSpec((_BM, 15), lambda i: (i, 0))],
        out_specs=[blkc(_HD),
                   pl.BlockSpec((_BM, 16), lambda i: (i, 0)),
                   pl.BlockSpec((_BM, 1), lambda i: (i, 0)),
                   pl.BlockSpec((_BM, 1), lambda i: (i, 0))],
        out_shape=[jax.ShapeDtypeStruct((_NC, _NPAD, _HD), _f32),
                   jax.ShapeDtypeStruct((_NPAD, 16), _f32),
                   jax.ShapeDtypeStruct((_NPAD, 1), _f32),
                   jax.ShapeDtypeStruct((_NPAD, 1), _f32)],
    )(degp, degl, x_p, x_l)


def _mid_tc_body(ap_ref, yp_ref, invp_ref, g_ref,
                 al_ref, yl_ref, invl_ref,
                 wp1_ref, bp1_ref, wp2_ref, bp2_ref, wl1_ref, bl1_ref,
                 y2_ref, zp_ref, sacc_ref):
    i = pl.program_id(0)
    invp = invp_ref[...]
    aggp = invp * jnp.concatenate(
        [ap_ref[0] + yp_ref[0], ap_ref[1] + yp_ref[1]], axis=1)
    h = jnp.maximum(
        jnp.dot(aggp, wp1_ref[...], preferred_element_type=_f32)
        + bp1_ref[...], 0.0)
    rowid = lax.broadcasted_iota(jnp.int32, (_BM, 1), 0) + i * _BM
    cvec = jnp.where(rowid < _N, invp * (g_ref[...] + invp), 0.0)

    @pl.when(i == 0)
    def _():
        sacc_ref[...] = jnp.zeros_like(sacc_ref)

    sacc_ref[...] += jnp.sum(cvec * h, axis=0, keepdims=True)

    invl = invl_ref[...]
    aggl = invl * (al_ref[0] + al_ref[1] + yl_ref[...])
    wl1p = jnp.concatenate([wl1_ref[...], jnp.zeros((1, 128), _f32)],
                           axis=0)
    hl = jnp.maximum(
        jnp.dot(aggl, wl1p, preferred_element_type=_f32)
        + bl1_ref[...], 0.0)
    y2 = invl * hl
    y2_ref[0, :, :] = y2[:, :_HD]
    y2_ref[1, :, :] = y2[:, _HD:]

    @pl.when(i == _GRID - 1)
    def _():
        zp = jnp.dot(sacc_ref[...] / _N, wp2_ref[...],
                     preferred_element_type=_f32) + bp2_ref[...]
        zp_ref[...] = zp[0]


def _mid_tc_call(aggp, yp3, invp, gmat, aggl, yl, invl,
                 Wp1, bp1, Wp2, bp2, Wl1, bl1):
    blk = lambda r, c: pl.BlockSpec((r, c), lambda i: (i, 0))
    blk3 = lambda c: pl.BlockSpec((_NC, _BM, c), lambda i: (0, i, 0))
    cst = lambda r, c: pl.BlockSpec((r, c), lambda i: (0, 0))
    return pl.pallas_call(
        _mid_tc_body,
        grid=(_GRID,),
        in_specs=[blk3(_HD), blk3(_HD), blk(_BM, 1), blk(_BM, 1),
                  blk3(16), blk(_BM, 16), blk(_BM, 1),
                  cst(128, 128), cst(1, 128), cst(128, 128), cst(1, 128),
                  cst(15, 128), cst(1, 128)],
        out_specs=[blk3(_HD),
                   pl.BlockSpec((128,), lambda i: (0,))],
        out_shape=[jax.ShapeDtypeStruct((_NC, _NPAD, _HD), _f32),
                   jax.ShapeDtypeStruct((128,), _f32)],
        scratch_shapes=[pltpu.VMEM((1, 128), _f32)],
    )(aggp, yp3, invp, gmat, aggl, yl, invl,
      Wp1, bp1[None, :], Wp2, bp2[None, :], Wl1, bl1[None, :])


def _dec_body(bfs_ref, xl_ref, xlf_ref,
              wd1_ref, bd1_ref, wd2_ref, bd2_ref,
              zv_ref, ht_ref, zvsum_ref, labB_ref):
    i = pl.program_id(0)
    rowid = lax.broadcasted_iota(jnp.int32, (_BM, 1), 0) + i * _BM

    @pl.when(i == 0)
    def _():
        zvsum_ref[...] = jnp.zeros_like(zvsum_ref)
        labB_ref[...] = jnp.zeros_like(labB_ref)

    stop_row = jnp.where(
        lax.broadcasted_iota(jnp.int32, (1, _NATOM), 1) == _NATOM - 1,
        1.0, 0.0)
    lab = jnp.where(rowid < _N, xl_ref[...][:, 4:], 0.0)
    lab = jnp.where(rowid == _N, stop_row, lab)
    labB_ref[...] += jnp.sum(jnp.where(rowid < _ND, lab, 0.0), axis=0,
                             keepdims=True)

    src0 = bfs_ref[0, 0]
    dst0 = bfs_ref[1, 0]
    same = src0 == dst0
    isq = 0.70710678118654752
    x_s0 = xlf_ref[pl.ds(src0, 1), :][:, 4:]
    x_d0 = xlf_ref[pl.ds(dst0, 1), :][:, 4:]
    agg_d = jnp.where(same, x_d0, isq * x_s0 + 0.5 * x_d0)
    h_spec = jnp.maximum(
        jnp.dot(agg_d, wd1_ref[...], preferred_element_type=_f32)
        + bd1_ref[...], 0.0)
    h_src0 = jnp.maximum(
        jnp.dot(x_s0, wd1_ref[...], preferred_element_type=_f32)
        + bd1_ref[...], 0.0)
    agg2_d = jnp.where(same, h_spec, isq * h_src0 + 0.5 * h_spec)
    z_spec = jnp.dot(agg2_d, wd2_ref[...], preferred_element_type=_f32) \
        + bd2_ref[...]
    h = jnp.maximum(
        jnp.dot(lab, wd1_ref[...], preferred_element_type=_f32)
        + bd1_ref[...], 0.0)
    h = jnp.where(rowid == dst0, h_spec, h)
    zv = jnp.dot(h, wd2_ref[...], preferred_element_type=_f32) \
        + bd2_ref[...]
    zv = jnp.where(rowid == dst0, z_spec, zv)
    zv_ref[...] = zv
    zvsum_ref[...] += jnp.sum(jnp.where(rowid < _ND, zv, 0.0), axis=0,
                              keepdims=True)

    @pl.when(i == _GRID - 1)
    def _():
        ht = jnp.concatenate([zvsum_ref[...] / _ND, labB_ref[...] / _ND],
                             axis=1)
        ht_ref[...] = ht[0]


def _dec_call(bfs, x_l, Wd1, bd1, Wd2, bd2):
    blk = lambda r, c: pl.BlockSpec((r, c), lambda i: (i, 0))
    cst = lambda r, c: pl.BlockSpec((r, c), lambda i: (0, 0))
    return pl.pallas_call(
        _dec_body,
        grid=(_GRID,),
        in_specs=[pl.BlockSpec(memory_space=pltpu.SMEM),
                  blk(_BM, 15),
                  cst(_N, 15),
                  cst(_NATOM, 128), cst(1, 128), cst(128, 128),
                  cst(1, 128)],
        out_specs=[blk(_BM, 128),
                   pl.BlockSpec((139,), lambda i: (0,))],
        out_shape=[jax.ShapeDtypeStruct((_ND, 128), _f32),
                   jax.ShapeDtypeStruct((139,), _f32)],
        scratch_shapes=[pltpu.VMEM((1, 128), _f32),
                        pltpu.VMEM((1, _NATOM), _f32)],
    )(bfs, x_l, x_l, Wd1, bd1[None, :], Wd2, bd2[None, :])


def _lig_body(q_ref, y2_ref, invl_ref, xl_ref,
              wl2_ref, bl2_ref, wf_ref, bf_ref,
              lp_ref, hi_ref, zlsum_ref, labA_ref):
    i = pl.program_id(0)
    rowid = lax.broadcasted_iota(jnp.int32, (_BM, 1), 0) + i * _BM

    @pl.when(i == 0)
    def _():
        lp_ref[...] = jnp.zeros_like(lp_ref)
        zlsum_ref[...] = jnp.zeros_like(zlsum_ref)
        labA_ref[...] = jnp.zeros_like(labA_ref)

    lab = jnp.where(rowid < _N, xl_ref[...][:, 4:], 0.0)
    invl = invl_ref[...]
    aggl2 = invl * jnp.concatenate(
        [q_ref[0] + y2_ref[0], q_ref[1] + y2_ref[1]], axis=1)
    zl = jnp.dot(aggl2, wl2_ref[...], preferred_element_type=_f32) \
        + bl2_ref[...]
    lmask = jnp.where(
        lax.broadcasted_iota(jnp.int32, (1, _NATOM), 1) == _NATOM - 1,
        -1e9, 0.0)
    logits = jnp.dot(zl, wf_ref[...], preferred_element_type=_f32) \
        + bf_ref[...] + lmask
    m = jnp.max(logits, axis=1, keepdims=True)
    e = jnp.exp(logits - m)
    num = jnp.sum(e * lab, axis=1, keepdims=True)
    den = jnp.sum(e, axis=1, keepdims=True)
    lig_mask = rowid < _N
    inner = jnp.where(lig_mask, num / den, 1.0)
    lp_ref[...] += jnp.sum(jnp.log(inner), axis=0, keepdims=True)
    zlsum_ref[...] += jnp.sum(jnp.where(lig_mask, zl, 0.0), axis=0,
                              keepdims=True)
    labA_ref[...] += jnp.sum(jnp.where(lig_mask, lab, 0.0), axis=0,
                             keepdims=True)

    @pl.when(i == _GRID - 1)
    def _():
        hi = jnp.concatenate([zlsum_ref[...] / _N, labA_ref[...] / _N],
                             axis=1)
        hi_ref[...] = hi[0]


def _lig_call(aggl2, y23, invl, x_l, Wl2, bl2, Wf, bf):
    blk = lambda r, c: pl.BlockSpec((r, c), lambda i: (i, 0))
    blk3 = lambda c: pl.BlockSpec((_NC, _BM, c), lambda i: (0, i, 0))
    cst = lambda r, c: pl.BlockSpec((r, c), lambda i: (0, 0))
    return pl.pallas_call(
        _lig_body,
        grid=(_GRID,),
        in_specs=[blk3(_HD), blk3(_HD), blk(_BM, 1), blk(_BM, 15),
                  cst(128, 128), cst(1, 128), cst(128, _NATOM),
                  cst(1, _NATOM)],
        out_specs=[cst(1, 1),
                   pl.BlockSpec((139,), lambda i: (0,))],
        out_shape=[jax.ShapeDtypeStruct((1, 1), _f32),
                   jax.ShapeDtypeStruct((139,), _f32)],
        scratch_shapes=[pltpu.VMEM((1, 128), _f32),
                        pltpu.VMEM((1, _NATOM), _f32)],
    )(aggl2, y23, invl, x_l, Wl2, bl2[None, :], Wf, bf[None, :])


def kernel(x_p, edge_index_p, x_l, edge_index_l, bfs_init, Wp1, bp1, Wp2,
           bp2, Wl1, bl1, Wl2, bl2, Wd1, bd1, Wd2, bd2, Wf, bf):
    ep4 = edge_index_p.reshape(2, _NS, _NCHP, _CH)
    el4 = edge_index_l.reshape(2, _NS, _NCHP, _CH)

    z_v, H_t = _dec_call(bfs_init, x_l, Wd1, bd1, Wd2, bd2)

    dp, dl = _deg_call(ep4, el4)
    degp = (dp[0] + dp[1] + 1.0)[:, None]
    degl = (dl[0] + dl[1] + 1.0)[:, None]

    yp3, yl, invp, invl = _prep_call(degp, degl, x_p, x_l)

    aggp, gmat, aggl = _mid_call(yp3, ep4, invp.reshape(_NPAD), yl, el4)
    gcol = (gmat[0] + gmat[1])[:, None]

    y23, z_pocket = _mid_tc_call(aggp, yp3, invp, gcol, aggl, yl, invl,
                                 Wp1, bp1, Wp2, bp2, Wl1, bl1)

    (aggl2,) = _l2_call(y23, el4)

    lp, H_init = _lig_call(aggl2, y23, invl, x_l, Wl2, bl2, Wf, bf)

    return (lp[0, 0], z_pocket, z_v, H_init, H_t)
